# Initial kernel scaffold; baseline (speedup 1.0000x reference)
import jax
import jax.numpy as jnp
from jax.experimental import pallas as pl


def kernel(samples, tuple_mapping, hash_matrix, filters):
    raise NotImplementedError("write your pallas kernel here")


# speedup vs baseline: 524.2878x; 524.2878x over previous
import functools

import jax
import jax.numpy as jnp
from jax import lax
from jax.experimental import pallas as pl
from jax.experimental.pallas import tpu as pltpu
from jax.experimental.pallas import tpu_sc as plsc

E = 4096
C = 64
T = 16
NN = E // T
F = 2048
H = 4
B = 1024

EW = E // 32
FW = F // 32
NC, NS, L = 2, 16, 16
NW = NC * NS
CPW = C // NW
BCH = 256
NCH = B // BCH


def _sc_body(samples, tm, hm, filters, out, spT,
             stage1, stage2, chunkbuf, fpacked, tmbuf, hmbuf, lut,
             packbuf, resp):
    cid = lax.axis_index("c")
    sid = lax.axis_index("s")
    wid = sid * NC + cid
    c0 = wid * CPW
    iota = lax.iota(jnp.int32, L)
    iota32 = iota * 32

    def phase1(it, carry):
        g = sid * 2 + (it >> 3)
        q8 = it & 7
        pltpu.sync_copy(
            samples.at[pl.ds(q8 * 128, 128), pl.ds(g * 128, 128)], stage1)

        for wl in range(4):
            colbase = jnp.full((L,), wl * 32, jnp.int32)

            def bv_body(bv, carry2, colbase=colbase, wl=wl):
                rows = bv * L + iota
                word = jnp.zeros((L,), jnp.int32)
                for i in range(32):
                    bits = plsc.load_gather(stage1, [rows, colbase + i])
                    word = word | jnp.left_shift(bits, i)
                packbuf[pl.ds(wl * 128 + bv * L, L)] = word
                return carry2

            lax.fori_loop(0, 128 // L, bv_body, 0)

        dst0 = ((q8 >> 1) * (EW * BCH)) + ((q8 & 1) * 128)
        for wl in range(4):
            pltpu.sync_copy(
                packbuf.at[pl.ds(wl * 128, 128)],
                spT.at[pl.ds(dst0 + (g * 4 + wl) * BCH, 128)])
        return carry

    lax.fori_loop(0, 2 * 8, phase1, 0)

    def phase2(it, carry):
        cl = it >> 5
        ch = it & 31
        n0 = ch * 8
        pltpu.sync_copy(filters.at[c0 + cl, pl.ds(n0, 8)], stage2)

        def n_body(n, carry2):
            nv = jnp.full((L,), n, jnp.int32)
            fbase = (cl * NN + n0 + n) * FW

            def wv_body(wv, carry3):
                word = jnp.zeros((L,), jnp.int32)
                colb = iota32 + wv * (32 * L)
                for i in range(32):
                    bits = plsc.load_gather(stage2, [nv, colb + i])
                    word = word | jnp.left_shift(bits, i)
                fpacked[pl.ds(fbase + wv * L, L)] = word
                return carry3

            lax.fori_loop(0, FW // L, wv_body, 0)
            return carry2

        lax.fori_loop(0, 8, n_body, 0)
        return carry

    lax.fori_loop(0, CPW * 32, phase2, 0)

    pltpu.sync_copy(hm, hmbuf)
    hj01 = hmbuf[0, :] | jnp.left_shift(hmbuf[1, :], 16)
    hj23 = hmbuf[2, :] | jnp.left_shift(hmbuf[3, :], 16)
    for hjv, jb, off in ((hj01, 0, 0), (hj01, 8, 256),
                         (hj23, 0, 512), (hj23, 8, 768)):
        hj = [hjv[jb + j] for j in range(8)]

        def xv_body(xv, carry, hj=hj, off=off):
            x = xv * L + iota
            a = jnp.zeros((L,), jnp.int32)
            for j in range(8):
                bit = jnp.right_shift(x, j) & 1
                a = a ^ ((0 - bit) & hj[j])
            lut[pl.ds(off + xv * L, L)] = a
            return carry

        lax.fori_loop(0, 256 // L, xv_body, 0)

    plsc.subcore_barrier()

    pltpu.sync_copy(tm.at[pl.ds(c0 * E, CPW * E)], tmbuf)
    for chunk in range(NCH):
        pltpu.sync_copy(spT.at[pl.ds(chunk * (EW * BCH), EW * BCH)], chunkbuf)
        for cl in range(CPW):
            def bv_body(bv, carry, cl=cl, chunk=chunk):
                bloc = bv * L

                def n_body(n, racc):
                    tmv = tmbuf[pl.ds(cl * E + n * T, T)]
                    fb = jnp.full((L,), (cl * NN + n) * FW, jnp.int32)
                    lo = jnp.zeros((L,), jnp.int32)
                    hi = jnp.zeros((L,), jnp.int32)
                    for j in range(8):
                        e = tmv[j]
                        v = chunkbuf[pl.ds(
                            jnp.left_shift(jnp.right_shift(e, 5), 8) + bloc,
                            L)]
                        lo = lo | jnp.left_shift(
                            jnp.right_shift(v, e & 31) & 1, j)
                        e2 = tmv[8 + j]
                        v2 = chunkbuf[pl.ds(
                            jnp.left_shift(jnp.right_shift(e2, 5), 8) + bloc,
                            L)]
                        hi = hi | jnp.left_shift(
                            jnp.right_shift(v2, e2 & 31) & 1, j)
                    h01 = (plsc.load_gather(lut, [lo])
                           ^ plsc.load_gather(lut, [hi + 256]))
                    h23 = (plsc.load_gather(lut, [lo + 512])
                           ^ plsc.load_gather(lut, [hi + 768]))
                    m = None
                    for hv in (h01 & 0xFFFF, jnp.right_shift(h01, 16),
                               h23 & 0xFFFF, jnp.right_shift(h23, 16)):
                        wrd = plsc.load_gather(
                            fpacked, [fb + jnp.right_shift(hv, 5)])
                        bitv = jnp.right_shift(wrd, hv & 31)
                        m = bitv if m is None else (m & bitv)
                    return racc + (m & 1)

                r = lax.fori_loop(0, NN, n_body, jnp.zeros((L,), jnp.int32))
                resp[pl.ds(cl * B + chunk * BCH + bloc, L)] = r
                return carry

            lax.fori_loop(0, BCH // L, bv_body, 0)

    pltpu.sync_copy(resp, out.at[pl.ds(wid * (CPW * B), CPW * B)])


_sc_kernel = functools.partial(
    pl.kernel,
    out_type=(jax.ShapeDtypeStruct((C * B,), jnp.int32),
              jax.ShapeDtypeStruct((NCH * EW * BCH,), jnp.int32)),
    mesh=plsc.VectorSubcoreMesh(
        core_axis_name="c", subcore_axis_name="s",
        num_cores=NC, num_subcores=NS),
    compiler_params=pltpu.CompilerParams(
        use_tc_tiling_on_sc=False, needs_layout_passes=False),
    scratch_types=[
        pltpu.VMEM((128, 128), jnp.int32),
        pltpu.VMEM((8, F), jnp.int32),
        pltpu.VMEM((EW * BCH,), jnp.int32),
        pltpu.VMEM((CPW * NN * FW,), jnp.int32),
        pltpu.VMEM((CPW * E,), jnp.int32),
        pltpu.VMEM((H, T), jnp.int32),
        pltpu.VMEM((1024,), jnp.int32),
        pltpu.VMEM((512,), jnp.int32),
        pltpu.VMEM((CPW * B,), jnp.int32),
    ],
)(_sc_body)


def kernel(samples, tuple_mapping, hash_matrix, filters):
    resp, _ = _sc_kernel(samples, tuple_mapping.reshape(-1), hash_matrix,
                         filters)
    return resp.reshape(C, B).T


# speedup vs baseline: 606.2511x; 1.1563x over previous
import functools

import jax
import jax.numpy as jnp
from jax import lax
from jax.experimental import pallas as pl
from jax.experimental.pallas import tpu as pltpu
from jax.experimental.pallas import tpu_sc as plsc

E = 4096
C = 64
T = 16
NN = E // T
F = 2048
H = 4
B = 1024

FW = F // 32
NC, NS, L = 2, 16, 16
NW = NC * NS
CPW = C // NW
CB = 128
NCHK = B // CB
EPS = E // NS


def _sc_body(samples, tm, hm, filters, out, spT,
             stage1, stage2, chunkbuf, fpacked, tmbuf, hmbuf,
             lutL01, lutH01, lutL23, lutH23, rowbuf, resp):
    cid = lax.axis_index("c")
    sid = lax.axis_index("s")
    wid = sid * NC + cid
    c0 = wid * CPW
    iota = lax.iota(jnp.int32, L)
    iota32 = iota * 32

    def phase1(chunk, carry):
        for eg in range(8):
            pltpu.sync_copy(
                samples.at[pl.ds(chunk * CB, CB),
                           pl.ds(sid * EPS + eg * 32, 32)], stage1)

            def el_body(el, c2):
                colv = jnp.full((L,), el, jnp.int32)
                word = jnp.zeros((L,), jnp.int32)
                for p in range(8):
                    bits = plsc.load_gather(stage1, [p * 16 + iota, colv])
                    word = word | jnp.left_shift(bits, 4 * p)
                rowbuf[pl.ds(el * 64, 64)] = plsc.bitcast(word, jnp.int8)
                return c2

            lax.fori_loop(0, 32, el_body, 0)
            pltpu.sync_copy(
                rowbuf,
                spT.at[pl.ds(chunk * (E * 64) + sid * (EPS * 64) + eg * 2048,
                             2048)])
        return carry

    lax.fori_loop(0, NCHK, phase1, 0)

    def phase2(it, carry):
        cl = it >> 5
        ch = it & 31
        n0 = ch * 8
        pltpu.sync_copy(filters.at[c0 + cl, pl.ds(n0, 8)], stage2)

        def n_body(n, carry2):
            nv = jnp.full((L,), n, jnp.int32)
            fbase = (cl * NN + n0 + n) * FW

            def wv_body(wv, carry3):
                word = jnp.zeros((L,), jnp.int32)
                colb = iota32 + wv * (32 * L)
                for i in range(32):
                    bits = plsc.load_gather(stage2, [nv, colb + i])
                    word = word | jnp.left_shift(bits, i)
                fpacked[pl.ds(fbase + wv * L, L)] = word
                return carry3

            lax.fori_loop(0, FW // L, wv_body, 0)
            return carry2

        lax.fori_loop(0, 8, n_body, 0)
        return carry

    lax.fori_loop(0, CPW * 32, phase2, 0)

    pltpu.sync_copy(hm, hmbuf)
    hj01 = hmbuf[0, :] | jnp.left_shift(hmbuf[1, :], 16)
    hj23 = hmbuf[2, :] | jnp.left_shift(hmbuf[3, :], 16)
    for ref, hjv, jb in ((lutL01, hj01, 0), (lutH01, hj01, 8),
                         (lutL23, hj23, 0), (lutH23, hj23, 8)):
        hj = [hjv[jb + j] for j in range(8)]

        def xv_body(xv, carry, hj=hj, ref=ref):
            x = xv * L + iota
            a = jnp.zeros((L,), jnp.int32)
            for j in range(8):
                bit = jnp.right_shift(x, j) & 1
                a = a ^ ((0 - bit) & hj[j])
            ref[pl.ds(xv * L, L)] = a
            return carry

        lax.fori_loop(0, 256 // L, xv_body, 0)

    plsc.subcore_barrier()

    def half_members(loX, hiX, fb):
        h01 = (plsc.load_gather(lutL01, [loX])
               ^ plsc.load_gather(lutH01, [hiX]))
        h23 = (plsc.load_gather(lutL23, [loX])
               ^ plsc.load_gather(lutH23, [hiX]))
        m = None
        for h2p in (h01, h23):
            w0 = plsc.load_gather(
                fpacked, [fb + (jnp.right_shift(h2p, 5) & 0x3F)])
            b0 = jnp.right_shift(w0, h2p & 31)
            w1 = plsc.load_gather(fpacked, [fb + jnp.right_shift(h2p, 21)])
            b1 = jnp.right_shift(w1, jnp.right_shift(h2p, 16) & 31)
            mb = b0 & b1
            m = mb if m is None else (m & mb)
        return m & 1

    c11 = jnp.full((64,), 0x11, jnp.int8)

    def phase4(it, carry):
        chunk = it >> 1
        cl = it & 1
        pltpu.sync_copy(spT.at[pl.ds(chunk * (E * 64), E * 64)], chunkbuf)
        pltpu.sync_copy(tm.at[pl.ds((c0 + cl) * E, E)], tmbuf)
        fb0 = cl * (NN * FW)

        def n_body(n, racc):
            tmv = tmbuf[pl.ds(n * T, T)]
            fb = jnp.full((L,), fb0 + n * FW, jnp.int32)
            accs = []
            for q in range(4):
                a = jnp.zeros((64,), jnp.int8)
                for jj in (3, 2, 1, 0):
                    e = tmv[q * 4 + jj]
                    v = chunkbuf[pl.ds(jnp.left_shift(e, 6), 64)]
                    a = a + a + (v & c11)
                accs.append(plsc.bitcast(a, jnp.int32))
            outs = []
            for p in range(8):
                sh = 4 * p
                lo = ((jnp.right_shift(accs[0], sh) & 0xF)
                      | jnp.left_shift(
                          jnp.right_shift(accs[1], sh) & 0xF, 4))
                hi = ((jnp.right_shift(accs[2], sh) & 0xF)
                      | jnp.left_shift(
                          jnp.right_shift(accs[3], sh) & 0xF, 4))
                outs.append(racc[p] + half_members(lo, hi, fb))
            return tuple(outs)

        z = jnp.zeros((L,), jnp.int32)
        racc = lax.fori_loop(0, NN, n_body, (z,) * 8)
        rbase = cl * B + chunk * CB
        for p in range(8):
            resp[pl.ds(rbase + p * L, L)] = racc[p]
        return carry

    lax.fori_loop(0, NCHK * CPW, phase4, 0)

    pltpu.sync_copy(resp, out.at[pl.ds(wid * (CPW * B), CPW * B)])


_sc_kernel = functools.partial(
    pl.kernel,
    out_type=(jax.ShapeDtypeStruct((C * B,), jnp.int32),
              jax.ShapeDtypeStruct((NCHK * E * 64,), jnp.int8)),
    mesh=plsc.VectorSubcoreMesh(
        core_axis_name="c", subcore_axis_name="s",
        num_cores=NC, num_subcores=NS),
    compiler_params=pltpu.CompilerParams(
        use_tc_tiling_on_sc=False, needs_layout_passes=False),
    scratch_types=[
        pltpu.VMEM((CB, 32), jnp.int32),
        pltpu.VMEM((8, F), jnp.int32),
        pltpu.VMEM((E * 64,), jnp.int8),
        pltpu.VMEM((CPW * NN * FW,), jnp.int32),
        pltpu.VMEM((E,), jnp.int32),
        pltpu.VMEM((H, T), jnp.int32),
        pltpu.VMEM((256,), jnp.int32),
        pltpu.VMEM((256,), jnp.int32),
        pltpu.VMEM((256,), jnp.int32),
        pltpu.VMEM((256,), jnp.int32),
        pltpu.VMEM((2048,), jnp.int8),
        pltpu.VMEM((CPW * B,), jnp.int32),
    ],
)(_sc_body)


def kernel(samples, tuple_mapping, hash_matrix, filters):
    resp, _ = _sc_kernel(samples, tuple_mapping.reshape(-1), hash_matrix,
                         filters)
    return resp.reshape(C, B).T


# speedup vs baseline: 629.0817x; 1.0377x over previous
import functools

import jax
import jax.numpy as jnp
from jax import lax
from jax.experimental import pallas as pl
from jax.experimental.pallas import tpu as pltpu
from jax.experimental.pallas import tpu_sc as plsc

E = 4096
C = 64
T = 16
NN = E // T
F = 2048
H = 4
B = 1024

FW = F // 32
NC, NS, L = 2, 16, 16
NW = NC * NS
CPW = C // NW
CB = 128
NCHK = B // CB
EPS = E // NS


def _sc_body(samples, tm, hm, filters, out, spT,
             stage1, stage2, chunkbuf, fpacked, tmbuf, hmbuf,
             lutL01, lutH01, lutL23, lutH23, rowbuf, resp):
    cid = lax.axis_index("c")
    sid = lax.axis_index("s")
    wid = sid * NC + cid
    c0 = wid * CPW
    iota = lax.iota(jnp.int32, L)
    iota32 = iota * 32

    def phase1(chunk, carry):
        for eg in range(8):
            pltpu.sync_copy(
                samples.at[pl.ds(chunk * CB, CB),
                           pl.ds(sid * EPS + eg * 32, 32)], stage1)

            def el_body(el, c2):
                colv = jnp.full((L,), el, jnp.int32)
                word = jnp.zeros((L,), jnp.int32)
                for p in range(8):
                    bits = plsc.load_gather(stage1, [p * 16 + iota, colv])
                    word = word | jnp.left_shift(bits, 4 * p)
                rowbuf[pl.ds(el * 64, 64)] = plsc.bitcast(word, jnp.int8)
                return c2

            lax.fori_loop(0, 32, el_body, 0)
            pltpu.sync_copy(
                rowbuf,
                spT.at[pl.ds(chunk * (E * 64) + sid * (EPS * 64) + eg * 2048,
                             2048)])
        return carry

    lax.fori_loop(0, NCHK, phase1, 0)

    def phase2(it, carry):
        cl = it >> 5
        ch = it & 31
        n0 = ch * 8
        pltpu.sync_copy(filters.at[c0 + cl, pl.ds(n0, 8)], stage2)

        def n_body(n, carry2):
            nv = jnp.full((L,), n, jnp.int32)
            fbase = (cl * NN + n0 + n) * FW

            def wv_body(wv, carry3):
                word = jnp.zeros((L,), jnp.int32)
                colb = iota32 + wv * (32 * L)
                for i in range(32):
                    bits = plsc.load_gather(stage2, [nv, colb + i])
                    word = word | jnp.left_shift(bits, i)
                fpacked[pl.ds(fbase + wv * L, L)] = word
                return carry3

            lax.fori_loop(0, FW // L, wv_body, 0)
            return carry2

        lax.fori_loop(0, 8, n_body, 0)
        return carry

    lax.fori_loop(0, CPW * 32, phase2, 0)

    pltpu.sync_copy(hm, hmbuf)
    hj01 = hmbuf[0, :] | jnp.left_shift(hmbuf[1, :], 16)
    hj23 = hmbuf[2, :] | jnp.left_shift(hmbuf[3, :], 16)
    for ref, hjv, jb in ((lutL01, hj01, 0), (lutH01, hj01, 8),
                         (lutL23, hj23, 0), (lutH23, hj23, 8)):
        hj = [hjv[jb + j] for j in range(8)]

        def xv_body(xv, carry, hj=hj, ref=ref):
            x = xv * L + iota
            a = jnp.zeros((L,), jnp.int32)
            for j in range(8):
                bit = jnp.right_shift(x, j) & 1
                a = a ^ ((0 - bit) & hj[j])
            ref[pl.ds(xv * L, L)] = a
            return carry

        lax.fori_loop(0, 256 // L, xv_body, 0)

    plsc.subcore_barrier()

    def half_members(loX, hiX, fb):
        h01 = (plsc.load_gather(lutL01, [loX])
               ^ plsc.load_gather(lutH01, [hiX]))
        h23 = (plsc.load_gather(lutL23, [loX])
               ^ plsc.load_gather(lutH23, [hiX]))
        m = None
        for h2p in (h01, h23):
            w0 = plsc.load_gather(
                fpacked, [fb + (jnp.right_shift(h2p, 5) & 0x3F)])
            b0 = jnp.right_shift(w0, h2p & 31)
            w1 = plsc.load_gather(fpacked, [fb + jnp.right_shift(h2p, 21)])
            b1 = jnp.right_shift(w1, jnp.right_shift(h2p, 16) & 31)
            mb = b0 & b1
            m = mb if m is None else (m & mb)
        return m & 1

    c11 = jnp.full((64,), 0x11, jnp.int8)

    def phase4(chunk, carry):
        pltpu.sync_copy(spT.at[pl.ds(chunk * (E * 64), E * 64)], chunkbuf)
        for cl in range(CPW):
            pltpu.sync_copy(tm.at[pl.ds((c0 + cl) * E, E)], tmbuf)
            fb0 = cl * (NN * FW)

            def n_body(n, racc, fb0=fb0):
                tmv = tmbuf[pl.ds(n * T, T)]
                fb = jnp.full((L,), fb0 + n * FW, jnp.int32)
                accs = []
                for q in range(4):
                    a = jnp.zeros((64,), jnp.int8)
                    for jj in (3, 2, 1, 0):
                        e = tmv[q * 4 + jj]
                        v = chunkbuf[pl.ds(jnp.left_shift(e, 6), 64)]
                        a = a + a + (v & c11)
                    accs.append(plsc.bitcast(a, jnp.int32))
                outs = []
                for p in range(8):
                    sh = 4 * p
                    lo = ((jnp.right_shift(accs[0], sh) & 0xF)
                          | jnp.left_shift(
                              jnp.right_shift(accs[1], sh) & 0xF, 4))
                    hi = ((jnp.right_shift(accs[2], sh) & 0xF)
                          | jnp.left_shift(
                              jnp.right_shift(accs[3], sh) & 0xF, 4))
                    outs.append(racc[p] + half_members(lo, hi, fb))
                return tuple(outs)

            z = jnp.zeros((L,), jnp.int32)
            racc = lax.fori_loop(0, NN, n_body, (z,) * 8)
            rbase = cl * B + chunk * CB
            for p in range(8):
                resp[pl.ds(rbase + p * L, L)] = racc[p]
        return carry

    lax.fori_loop(0, NCHK, phase4, 0)

    pltpu.sync_copy(resp, out.at[pl.ds(wid * (CPW * B), CPW * B)])


_sc_kernel = functools.partial(
    pl.kernel,
    out_type=(jax.ShapeDtypeStruct((C * B,), jnp.int32),
              jax.ShapeDtypeStruct((NCHK * E * 64,), jnp.int8)),
    mesh=plsc.VectorSubcoreMesh(
        core_axis_name="c", subcore_axis_name="s",
        num_cores=NC, num_subcores=NS),
    compiler_params=pltpu.CompilerParams(
        use_tc_tiling_on_sc=False, needs_layout_passes=False),
    scratch_types=[
        pltpu.VMEM((CB, 32), jnp.int32),
        pltpu.VMEM((8, F), jnp.int32),
        pltpu.VMEM((E * 64,), jnp.int8),
        pltpu.VMEM((CPW * NN * FW,), jnp.int32),
        pltpu.VMEM((E,), jnp.int32),
        pltpu.VMEM((H, T), jnp.int32),
        pltpu.VMEM((256,), jnp.int32),
        pltpu.VMEM((256,), jnp.int32),
        pltpu.VMEM((256,), jnp.int32),
        pltpu.VMEM((256,), jnp.int32),
        pltpu.VMEM((2048,), jnp.int8),
        pltpu.VMEM((CPW * B,), jnp.int32),
    ],
)(_sc_body)


def kernel(samples, tuple_mapping, hash_matrix, filters):
    resp, _ = _sc_kernel(samples, tuple_mapping.reshape(-1), hash_matrix,
                         filters)
    return resp.reshape(C, B).T
